# baseline (device time: 79657 ns/iter reference)
import os

import jax
import jax.numpy as jnp
from jax import lax
from jax.experimental import pallas as pl
from jax.experimental.pallas import tpu as pltpu

if "KNCHUNK" in os.environ:
    _N = int(os.environ["KNCHUNK"])
    CHUNKS = (2048 // _N,) * _N
else:
    CHUNKS = (128, 256, 512, 512, 384, 128, 128)
VARIANT = os.environ.get("KVARIANT", "full")
DO_COMPUTE = "nocompute" not in VARIANT
DO_YCOMM = VARIANT != "nocomm" and "noycomm" not in VARIANT
DO_XCOMM = VARIANT != "nocomm" and "noxcomm" not in VARIANT


def kernel(x, dy):
    m, d = x.shape
    _, f = dy.shape
    half_d = d // 2
    half_f = f // 2
    n_chunk = len(CHUNKS)
    offs = [sum(CHUNKS[:i]) for i in range(n_chunk)]
    assert sum(CHUNKS) == half_f

    def body(x_ref, dy_ref, out_ref, xt_ref, p_ref, yrecv_ref,
             y_send_sems, y_recv_sems, x_send_sems, x_recv_sems):
        my_x = lax.axis_index("x")
        my_y = lax.axis_index("y")
        col0 = my_x * half_f
        ocol0 = (1 - my_x) * half_f

        barrier_sem = pltpu.get_barrier_semaphore()
        pl.semaphore_signal(barrier_sem, inc=1,
                            device_id=(my_x, 1 - my_y),
                            device_id_type=pl.DeviceIdType.MESH)
        pl.semaphore_signal(barrier_sem, inc=1,
                            device_id=(1 - my_x, my_y),
                            device_id_type=pl.DeviceIdType.MESH)
        pl.semaphore_wait(barrier_sem, 2)

        other_rows = pl.ds((1 - my_y) * half_d, half_d)
        my_rows = pl.ds(my_y * half_d, half_d)

        if DO_COMPUTE:
            xt_ref[other_rows, :] = x_ref[:, other_rows].T
        y_rdmas = []
        for c in range(n_chunk):
            cf, off = CHUNKS[c], offs[c]
            if DO_COMPUTE:
                p_ref[other_rows, pl.ds(off, cf)] = lax.dot_general(
                    xt_ref[other_rows, :],
                    dy_ref[:, pl.ds(col0 + off, cf)],
                    (((1,), (0,)), ((), ())),
                    preferred_element_type=jnp.float32,
                )
            if not DO_YCOMM:
                continue
            rdma_y = pltpu.make_async_remote_copy(
                src_ref=p_ref.at[other_rows, pl.ds(off, cf)],
                dst_ref=yrecv_ref.at[:, pl.ds(off, cf)],
                send_sem=y_send_sems.at[c],
                recv_sem=y_recv_sems.at[c],
                device_id=(my_x, 1 - my_y),
                device_id_type=pl.DeviceIdType.MESH,
            )
            rdma_y.start()
            y_rdmas.append(rdma_y)

        if DO_COMPUTE:
            xt_ref[my_rows, :] = x_ref[:, my_rows].T
            for c in range(n_chunk):
                cf, off = CHUNKS[c], offs[c]
                p_ref[my_rows, pl.ds(off, cf)] = lax.dot_general(
                    xt_ref[my_rows, :],
                    dy_ref[:, pl.ds(col0 + off, cf)],
                    (((1,), (0,)), ((), ())),
                    preferred_element_type=jnp.float32,
                )

        x_rdmas = []
        for c in range(n_chunk):
            cf, off = CHUNKS[c], offs[c]
            if DO_YCOMM:
                y_rdmas[c].wait_recv()
                out_ref[:, pl.ds(col0 + off, cf)] = (
                    p_ref[my_rows, pl.ds(off, cf)]
                    + yrecv_ref[:, pl.ds(off, cf)]
                )
            else:
                out_ref[:, pl.ds(col0 + off, cf)] = (
                    p_ref[my_rows, pl.ds(off, cf)]
                    + p_ref[other_rows, pl.ds(off, cf)]
                )
            if not DO_XCOMM:
                out_ref[:, pl.ds(ocol0 + off, cf)] = out_ref[:, pl.ds(col0 + off, cf)]
                continue
            rdma_x = pltpu.make_async_remote_copy(
                src_ref=out_ref.at[:, pl.ds(col0 + off, cf)],
                dst_ref=out_ref.at[:, pl.ds(col0 + off, cf)],
                send_sem=x_send_sems.at[c],
                recv_sem=x_recv_sems.at[c],
                device_id=(1 - my_x, my_y),
                device_id_type=pl.DeviceIdType.MESH,
            )
            rdma_x.start()
            x_rdmas.append(rdma_x)

        for rdma_x in x_rdmas:
            rdma_x.wait_recv()
        for rdma_y in y_rdmas:
            rdma_y.wait_send()
        for rdma_x in x_rdmas:
            rdma_x.wait_send()

    return pl.pallas_call(
        body,
        out_shape=jax.ShapeDtypeStruct((half_d, f), jnp.float32),
        in_specs=[
            pl.BlockSpec(memory_space=pltpu.VMEM),
            pl.BlockSpec(memory_space=pltpu.VMEM),
        ],
        out_specs=pl.BlockSpec(memory_space=pltpu.VMEM),
        scratch_shapes=[
            pltpu.VMEM((d, m), jnp.float32),
            pltpu.VMEM((d, half_f), jnp.float32),
            pltpu.VMEM((half_d, half_f), jnp.float32),
            pltpu.SemaphoreType.DMA((n_chunk,)),
            pltpu.SemaphoreType.DMA((n_chunk,)),
            pltpu.SemaphoreType.DMA((n_chunk,)),
            pltpu.SemaphoreType.DMA((n_chunk,)),
        ],
        compiler_params=pltpu.CompilerParams(
            collective_id=0, vmem_limit_bytes=100 * 1024 * 1024
        ),
    )(x, dy)


# device time: 70997 ns/iter; 1.1220x vs baseline; 1.1220x over previous
import os

import jax
import jax.numpy as jnp
from jax import lax
from jax.experimental import pallas as pl
from jax.experimental.pallas import tpu as pltpu

_N = int(os.environ.get("KNCHUNK", "8"))
CHUNKS = (2048 // _N,) * _N
VARIANT = os.environ.get("KVARIANT", "full")
DO_COMPUTE = "nocompute" not in VARIANT
DO_YCOMM = VARIANT != "nocomm" and "noycomm" not in VARIANT
DO_XCOMM = VARIANT != "nocomm" and "noxcomm" not in VARIANT


def kernel(x, dy):
    m, d = x.shape
    _, f = dy.shape
    half_d = d // 2
    half_f = f // 2
    n_chunk = len(CHUNKS)
    offs = [sum(CHUNKS[:i]) for i in range(n_chunk)]
    assert sum(CHUNKS) == half_f

    def body(x_ref, dy_ref, out_ref, xt_ref, dyv_ref, p_ref, yrecv_ref,
             dyin_sems, store_sems,
             y_send_sems, y_recv_sems, x_send_sems, x_recv_sems):
        my_x = lax.axis_index("x")
        my_y = lax.axis_index("y")
        col0 = my_x * half_f

        barrier_sem = pltpu.get_barrier_semaphore()
        pl.semaphore_signal(barrier_sem, inc=1,
                            device_id=(my_x, 1 - my_y),
                            device_id_type=pl.DeviceIdType.MESH)
        pl.semaphore_signal(barrier_sem, inc=1,
                            device_id=(1 - my_x, my_y),
                            device_id_type=pl.DeviceIdType.MESH)
        pl.semaphore_wait(barrier_sem, 2)

        other_rows = pl.ds((1 - my_y) * half_d, half_d)
        my_rows = pl.ds(my_y * half_d, half_d)

        dyin = []
        for c in range(n_chunk):
            cf, off = CHUNKS[c], offs[c]
            cp = pltpu.make_async_copy(
                dy_ref.at[:, pl.ds(col0 + off, cf)],
                dyv_ref.at[:, pl.ds(off, cf)],
                dyin_sems.at[c],
            )
            cp.start()
            dyin.append(cp)

        if DO_COMPUTE:
            xt_ref[:, :] = x_ref[:, :].T
        y_rdmas = []
        for c in range(n_chunk):
            cf, off = CHUNKS[c], offs[c]
            dyin[c].wait()
            if DO_COMPUTE:
                p_ref[:, pl.ds(off, cf)] = lax.dot_general(
                    xt_ref[:, :],
                    dyv_ref[:, pl.ds(off, cf)],
                    (((1,), (0,)), ((), ())),
                    preferred_element_type=jnp.float32,
                )
            if not DO_YCOMM:
                continue
            rdma_y = pltpu.make_async_remote_copy(
                src_ref=p_ref.at[other_rows, pl.ds(off, cf)],
                dst_ref=yrecv_ref.at[:, pl.ds(off, cf)],
                send_sem=y_send_sems.at[c],
                recv_sem=y_recv_sems.at[c],
                device_id=(my_x, 1 - my_y),
                device_id_type=pl.DeviceIdType.MESH,
            )
            rdma_y.start()
            y_rdmas.append(rdma_y)

        x_rdmas = []
        stores = []
        for c in range(n_chunk):
            cf, off = CHUNKS[c], offs[c]
            if DO_YCOMM:
                y_rdmas[c].wait_recv()
                yrecv_ref[:, pl.ds(off, cf)] = (
                    p_ref[my_rows, pl.ds(off, cf)]
                    + yrecv_ref[:, pl.ds(off, cf)]
                )
            else:
                yrecv_ref[:, pl.ds(off, cf)] = (
                    p_ref[my_rows, pl.ds(off, cf)]
                    + p_ref[other_rows, pl.ds(off, cf)]
                )
            st = pltpu.make_async_copy(
                yrecv_ref.at[:, pl.ds(off, cf)],
                out_ref.at[:, pl.ds(col0 + off, cf)],
                store_sems.at[c],
            )
            st.start()
            stores.append(st)
            if not DO_XCOMM:
                continue
            rdma_x = pltpu.make_async_remote_copy(
                src_ref=yrecv_ref.at[:, pl.ds(off, cf)],
                dst_ref=out_ref.at[:, pl.ds(col0 + off, cf)],
                send_sem=x_send_sems.at[c],
                recv_sem=x_recv_sems.at[c],
                device_id=(1 - my_x, my_y),
                device_id_type=pl.DeviceIdType.MESH,
            )
            rdma_x.start()
            x_rdmas.append(rdma_x)

        for st in stores:
            st.wait()
        for rdma_x in x_rdmas:
            rdma_x.wait_recv()
        for rdma_y in y_rdmas:
            rdma_y.wait_send()
        for rdma_x in x_rdmas:
            rdma_x.wait_send()

    return pl.pallas_call(
        body,
        out_shape=jax.ShapeDtypeStruct((half_d, f), jnp.float32),
        in_specs=[
            pl.BlockSpec(memory_space=pltpu.VMEM),
            pl.BlockSpec(memory_space=pl.ANY),
        ],
        out_specs=pl.BlockSpec(memory_space=pl.ANY),
        scratch_shapes=[
            pltpu.VMEM((d, m), jnp.float32),
            pltpu.VMEM((m, half_f), jnp.float32),
            pltpu.VMEM((d, half_f), jnp.float32),
            pltpu.VMEM((half_d, half_f), jnp.float32),
            pltpu.SemaphoreType.DMA((n_chunk,)),
            pltpu.SemaphoreType.DMA((n_chunk,)),
            pltpu.SemaphoreType.DMA((n_chunk,)),
            pltpu.SemaphoreType.DMA((n_chunk,)),
            pltpu.SemaphoreType.DMA((n_chunk,)),
            pltpu.SemaphoreType.DMA((n_chunk,)),
        ],
        compiler_params=pltpu.CompilerParams(
            collective_id=0, vmem_limit_bytes=100 * 1024 * 1024
        ),
    )(x, dy)


# device time: 69848 ns/iter; 1.1404x vs baseline; 1.0165x over previous
import os

import jax
import jax.numpy as jnp
from jax import lax
from jax.experimental import pallas as pl
from jax.experimental.pallas import tpu as pltpu

_N = int(os.environ.get("KNCHUNK", "8"))
CHUNKS = (2048 // _N,) * _N
VARIANT = os.environ.get("KVARIANT", "full")
DO_COMPUTE = "nocompute" not in VARIANT
DO_YCOMM = VARIANT != "nocomm" and "noycomm" not in VARIANT
DO_XCOMM = VARIANT != "nocomm" and "noxcomm" not in VARIANT


def kernel(x, dy):
    m, d = x.shape
    _, f = dy.shape
    half_d = d // 2
    half_f = f // 2
    n_chunk = len(CHUNKS)
    offs = [sum(CHUNKS[:i]) for i in range(n_chunk)]
    assert sum(CHUNKS) == half_f

    def body(x_ref, dy_ref, out_ref, xt_ref, dyv_ref, p_ref, yrecv_ref,
             dyin_sems, store_sems,
             y_send_sems, y_recv_sems, x_send_sems, x_recv_sems):
        my_x = lax.axis_index("x")
        my_y = lax.axis_index("y")
        col0 = my_x * half_f

        barrier_sem = pltpu.get_barrier_semaphore()
        pl.semaphore_signal(barrier_sem, inc=1,
                            device_id=(my_x, 1 - my_y),
                            device_id_type=pl.DeviceIdType.MESH)
        pl.semaphore_signal(barrier_sem, inc=1,
                            device_id=(1 - my_x, my_y),
                            device_id_type=pl.DeviceIdType.MESH)

        other_rows = pl.ds((1 - my_y) * half_d, half_d)
        my_rows = pl.ds(my_y * half_d, half_d)

        dyin = []
        for c in range(n_chunk):
            cf, off = CHUNKS[c], offs[c]
            cp = pltpu.make_async_copy(
                dy_ref.at[:, pl.ds(col0 + off, cf)],
                dyv_ref.at[:, pl.ds(off, cf)],
                dyin_sems.at[c],
            )
            cp.start()
            dyin.append(cp)

        if DO_COMPUTE:
            xt_ref[:, :] = x_ref[:, :].T
        pl.semaphore_wait(barrier_sem, 2)
        y_rdmas = []
        for c in range(n_chunk):
            cf, off = CHUNKS[c], offs[c]
            dyin[c].wait()
            if DO_COMPUTE:
                p_ref[:, pl.ds(off, cf)] = lax.dot_general(
                    xt_ref[:, :],
                    dyv_ref[:, pl.ds(off, cf)],
                    (((1,), (0,)), ((), ())),
                    preferred_element_type=jnp.float32,
                )
            if not DO_YCOMM:
                continue
            rdma_y = pltpu.make_async_remote_copy(
                src_ref=p_ref.at[other_rows, pl.ds(off, cf)],
                dst_ref=yrecv_ref.at[:, pl.ds(off, cf)],
                send_sem=y_send_sems.at[c],
                recv_sem=y_recv_sems.at[c],
                device_id=(my_x, 1 - my_y),
                device_id_type=pl.DeviceIdType.MESH,
            )
            rdma_y.start()
            y_rdmas.append(rdma_y)

        x_rdmas = []
        stores = []
        for c in range(n_chunk):
            cf, off = CHUNKS[c], offs[c]
            if DO_YCOMM:
                y_rdmas[c].wait_recv()
                yrecv_ref[:, pl.ds(off, cf)] = (
                    p_ref[my_rows, pl.ds(off, cf)]
                    + yrecv_ref[:, pl.ds(off, cf)]
                )
            else:
                yrecv_ref[:, pl.ds(off, cf)] = (
                    p_ref[my_rows, pl.ds(off, cf)]
                    + p_ref[other_rows, pl.ds(off, cf)]
                )
            if DO_XCOMM:
                rdma_x = pltpu.make_async_remote_copy(
                    src_ref=yrecv_ref.at[:, pl.ds(off, cf)],
                    dst_ref=out_ref.at[:, pl.ds(col0 + off, cf)],
                    send_sem=x_send_sems.at[c],
                    recv_sem=x_recv_sems.at[c],
                    device_id=(1 - my_x, my_y),
                    device_id_type=pl.DeviceIdType.MESH,
                )
                rdma_x.start()
                x_rdmas.append(rdma_x)
            st = pltpu.make_async_copy(
                yrecv_ref.at[:, pl.ds(off, cf)],
                out_ref.at[:, pl.ds(col0 + off, cf)],
                store_sems.at[c],
            )
            st.start()
            stores.append(st)

        for st in stores:
            st.wait()
        for rdma_x in x_rdmas:
            rdma_x.wait_recv()
        for rdma_y in y_rdmas:
            rdma_y.wait_send()
        for rdma_x in x_rdmas:
            rdma_x.wait_send()

    return pl.pallas_call(
        body,
        out_shape=jax.ShapeDtypeStruct((half_d, f), jnp.float32),
        in_specs=[
            pl.BlockSpec(memory_space=pltpu.VMEM),
            pl.BlockSpec(memory_space=pl.ANY),
        ],
        out_specs=pl.BlockSpec(memory_space=pl.ANY),
        scratch_shapes=[
            pltpu.VMEM((d, m), jnp.float32),
            pltpu.VMEM((m, half_f), jnp.float32),
            pltpu.VMEM((d, half_f), jnp.float32),
            pltpu.VMEM((half_d, half_f), jnp.float32),
            pltpu.SemaphoreType.DMA((n_chunk,)),
            pltpu.SemaphoreType.DMA((n_chunk,)),
            pltpu.SemaphoreType.DMA((n_chunk,)),
            pltpu.SemaphoreType.DMA((n_chunk,)),
            pltpu.SemaphoreType.DMA((n_chunk,)),
            pltpu.SemaphoreType.DMA((n_chunk,)),
        ],
        compiler_params=pltpu.CompilerParams(
            collective_id=0, vmem_limit_bytes=100 * 1024 * 1024
        ),
    )(x, dy)
